# Initial kernel scaffold; baseline (speedup 1.0000x reference)
import jax
import jax.numpy as jnp
from jax.experimental import pallas as pl


def kernel(adj_indices, adj_values, conditions_values, pc_w1, pc_b1, pc_w2, pc_b2, cu_w1, cu_b1, cu_w2, cu_b2, vu_w1, vu_b1, vu_w2, vu_b2, out_w1, out_b1, out_w2, out_b2):
    raise NotImplementedError("write your pallas kernel here")


# speedup vs baseline: 10.7504x; 10.7504x over previous
import functools

import jax
import jax.numpy as jnp
import numpy as np
from jax import lax
from jax.experimental import pallas as pl
from jax.experimental.pallas import tpu as pltpu
from jax.experimental.pallas import tpu_sc as plsc

V = 16384
C = 16384
NNZ = 1048576
F = 64
OUT_BITS = 16
STEPS = 3

NC = 2
NS = 16
NW = NC * NS
ET = NNZ // NW
SUB = 128
CHROWS = 64
NCHUNK = ET // (CHROWS * SUB)
ROWS_PER_TILE = V // NS

def _lane_bcast(vec, lane):
    return lax.gather(
        vec, jnp.full((16, 1), lane, jnp.int32),
        lax.GatherDimensionNumbers(offset_dims=(), collapsed_slice_dims=(0,),
                                   start_index_map=(0,)),
        slice_sizes=(1,), mode=lax.GatherScatterMode.PROMISE_IN_BOUNDS)


def _spmm_kernel(xd_hbm, gidx_hbm, sidx_hbm, vals_hbm, out_hbm,
                 gbuf, sbuf, vbuf, rows, msg, ysh, gsem):
    c = lax.axis_index("c")
    s = lax.axis_index("s")
    w = c * NS + s

    zeros16 = jnp.zeros((16,), jnp.float32)
    for r in range(SUB):
        for q in range(F // 16):
            msg[r, pl.ds(16 * q, 16)] = zeros16
    iota16 = lax.iota(jnp.int32, 16)
    for k in range(ROWS_PER_TILE // SUB):
        base0 = s * ROWS_PER_TILE + k * SUB
        for g in range(SUB // 16):
            sbuf[0, 0, pl.ds(g * 16, 16)] = iota16 + (base0 + g * 16)
        pltpu.sync_copy(msg, ysh.at[sbuf.at[0, 0]])
    plsc.subcore_barrier()

    def _window(jj, carry):
        @pl.when(jj % CHROWS == 0)
        def _stage():
            r0 = pl.multiple_of(w * (ET // 128) + jj, CHROWS)
            pltpu.sync_copy(gidx_hbm.at[pl.ds(r0, CHROWS)], gbuf)
            pltpu.sync_copy(sidx_hbm.at[pl.ds(r0, CHROWS)], sbuf)
            pltpu.sync_copy(vals_hbm.at[pl.ds(r0, CHROWS)], vbuf)

        j = jj % CHROWS
        pltpu.async_copy(xd_hbm.at[gbuf.at[j, 0]], rows, gsem).wait()
        for g in range(SUB // 16):
            vv = vbuf[j, 0, pl.ds(g * 16, 16)]
            for t in range(16):
                e = g * 16 + t
                val = _lane_bcast(vv, t)
                for q in range(F // 16):
                    msg[e, pl.ds(16 * q, 16)] = (
                        rows[e, pl.ds(16 * q, 16)] * val)
        pltpu.sync_copy(msg, ysh.at[sbuf.at[j, 0]], add=True)
        return carry

    lax.fori_loop(0, ET // SUB, _window, None)
    plsc.subcore_barrier()

    for k in range(ROWS_PER_TILE // SUB):
        base0 = s * ROWS_PER_TILE + k * SUB
        for g in range(SUB // 16):
            sbuf[0, 0, pl.ds(g * 16, 16)] = iota16 + (base0 + g * 16)
        pltpu.async_copy(ysh.at[sbuf.at[0, 0]], msg, gsem).wait()
        obase = pl.multiple_of(c * V + base0, SUB)
        pltpu.sync_copy(msg, out_hbm.at[pl.ds(obase, SUB)])


def _make_spmm():
    mesh = plsc.VectorSubcoreMesh(core_axis_name="c", subcore_axis_name="s",
                                  num_cores=NC, num_subcores=NS)
    return pl.kernel(
        _spmm_kernel,
        out_type=jax.ShapeDtypeStruct((NC * V, F), jnp.float32),
        mesh=mesh,
        scratch_types=[
            pltpu.VMEM((CHROWS, 1, 128), jnp.int32),
            pltpu.VMEM((CHROWS, 1, 128), jnp.int32),
            pltpu.VMEM((CHROWS, 1, 128), jnp.float32),
            pltpu.VMEM((SUB, 2 * F), jnp.float32),
            pltpu.VMEM((SUB, F), jnp.float32),
            pltpu.VMEM_SHARED((V, F), jnp.float32),
            pltpu.SemaphoreType.DMA,
        ],
    )


BLK = 2048
NBLK = V // BLK


def _dot(a, b):
    return jax.lax.dot_general(a, b, (((1,), (0,)), ((), ())),
                               preferred_element_type=jnp.float32,
                               precision=jax.lax.Precision.HIGHEST)


def _acc_stats(i, x, stats_ref):
    @pl.when(i == 0)
    def _():
        stats_ref[...] = jnp.zeros_like(stats_ref)

    stats_ref[0:1, :] += jnp.sum(x, axis=0, keepdims=True)
    stats_ref[1:2, :] += jnp.sum(x * x, axis=0, keepdims=True)


def _emb_kernel(cond_ref, w1_ref, b1_ref, w2_ref, b2_ref, out_ref,
                stats_ref):
    h = cond_ref[...] * w1_ref[...] + b1_ref[...]
    h = jnp.maximum(h, 0.0)
    x = _dot(h, w2_ref[...]) + b2_ref[...]
    out_ref[...] = x
    _acc_stats(pl.program_id(0), x, stats_ref)


def _cu_kernel(cons_ref, emb_ref, p_ref, w1_ref, b1_ref, w2_ref,
               b2_ref, out_ref, stats_ref):
    v2c = p_ref[0] + p_ref[1]
    h = (_dot(cons_ref[...], w1_ref[0:F]) + _dot(emb_ref[...], w1_ref[F:2*F])
         + _dot(v2c, w1_ref[2*F:3*F]) + b1_ref[...])
    h = jnp.where(h > 0, h, 0.01 * h)
    x = _dot(h, w2_ref[...]) + b2_ref[...]
    out_ref[...] = x
    _acc_stats(pl.program_id(0), x, stats_ref)


def _vu_kernel(vars_ref, p_ref, w1_ref, b1_ref, w2_ref, b2_ref,
               out_ref, stats_ref):
    c2v = p_ref[0] + p_ref[1]
    h = (_dot(vars_ref[...], w1_ref[0:F]) + _dot(c2v, w1_ref[F:2*F])
         + b1_ref[...])
    h = jnp.maximum(h, 0.0)
    x = _dot(h, w2_ref[...]) + b2_ref[...]
    out_ref[...] = x
    _acc_stats(pl.program_id(0), x, stats_ref)


def _pn_finish(x, stats, n, eps=1e-6):
    mean = stats[0:1, :] / n
    sumsq = jnp.sum(stats[1:2, :]) / n - jnp.sum(mean * mean)
    norm = jnp.sqrt(eps + sumsq)
    return (x - mean) / norm


def _pn_kernel(x_ref, stats_ref, y_ref, xd_ref):
    y = _pn_finish(x_ref[...], stats_ref[...], V)
    y_ref[...] = y
    xd_ref[...] = jnp.concatenate([y, y], axis=-1)


def _pn_nodup_kernel(x_ref, stats_ref, y_ref):
    y_ref[...] = _pn_finish(x_ref[...], stats_ref[...], V)


def _out_kernel(vars_ref, stats_ref, ow1_ref, ob1_ref, ow2_ref, ob2_ref,
                noise_ref, bin_ref, dec_ref):
    variables = _pn_finish(vars_ref[...], stats_ref[...], V)
    o = jnp.maximum(_dot(variables, ow1_ref[...]) + ob1_ref[...], 0.0)
    o = _dot(o, ow2_ref[...]) + ob2_ref[...]
    out = jax.nn.sigmoid(o + noise_ref[...])
    bin_ref[...] = out
    powers = 2.0 ** lax.broadcasted_iota(jnp.int32, (1, OUT_BITS), 1
                                         ).astype(jnp.float32)
    dec_ref[...] = jnp.sum(powers * out, axis=-1, keepdims=True)


def _row_spec(cols):
    return pl.BlockSpec((BLK, cols), lambda i: (i, 0))


def _full_spec(r, c):
    return pl.BlockSpec((r, c), lambda i: (0, 0))


_STATS_SPEC = pl.BlockSpec((2, F), lambda i: (0, 0))
_STATS_SHAPE = jax.ShapeDtypeStruct((2, F), jnp.float32)
_P_SPEC = pl.BlockSpec((2, BLK, F), lambda i: (0, i, 0))


def kernel(adj_indices, adj_values, conditions_values,
           pc_w1, pc_b1, pc_w2, pc_b2,
           cu_w1, cu_b1, cu_w2, cu_b2,
           vu_w1, vu_b1, vu_w2, vu_b2,
           out_w1, out_b1, out_w2, out_b2):
    rows_idx = adj_indices[0].astype(jnp.int32).reshape(NNZ // 128, 1, 128)
    cols_idx = adj_indices[1].astype(jnp.int32).reshape(NNZ // 128, 1, 128)
    vals = adj_values.astype(jnp.float32).reshape(NNZ // 128, 1, 128)

    spmm = _make_spmm()

    cond = conditions_values.reshape(C, 1)
    emb_pre, emb_stats = pl.pallas_call(
        _emb_kernel,
        grid=(NBLK,),
        in_specs=[_row_spec(1), _full_spec(1, F), _full_spec(1, F),
                  _full_spec(F, F), _full_spec(1, F)],
        out_specs=[_row_spec(F), _STATS_SPEC],
        out_shape=[jax.ShapeDtypeStruct((C, F), jnp.float32), _STATS_SHAPE],
    )(cond, pc_w1, pc_b1.reshape(1, F), pc_w2, pc_b2.reshape(1, F))
    emb = pl.pallas_call(
        _pn_nodup_kernel,
        grid=(NBLK,),
        in_specs=[_row_spec(F), _STATS_SPEC],
        out_specs=[_row_spec(F)],
        out_shape=[jax.ShapeDtypeStruct((C, F), jnp.float32)],
    )(emb_pre, emb_stats)[0]

    noise_key = jax.random.key(42)
    noises = [jax.random.normal(jax.random.fold_in(noise_key, i),
                                (V, OUT_BITS), dtype=jnp.float32) * 3.0
              for i in range(STEPS)]

    constraints = emb
    variables = jnp.ones((V, F), jnp.float32)
    variables_xd = jnp.ones((V, 2 * F), jnp.float32)
    binary_outputs = []
    decimal_outputs = []
    for i in range(STEPS):
        p = spmm(variables_xd, rows_idx, cols_idx, vals).reshape(NC, C, F)
        cons_pre, cons_stats = pl.pallas_call(
            _cu_kernel,
            grid=(NBLK,),
            in_specs=[_row_spec(F), _row_spec(F), _P_SPEC,
                      _full_spec(3 * F, F), _full_spec(1, F),
                      _full_spec(F, F), _full_spec(1, F)],
            out_specs=[_row_spec(F), _STATS_SPEC],
            out_shape=[jax.ShapeDtypeStruct((C, F), jnp.float32),
                       _STATS_SHAPE],
        )(constraints, emb, p, cu_w1, cu_b1.reshape(1, F),
          cu_w2, cu_b2.reshape(1, F))
        constraints, constraints_xd = pl.pallas_call(
            _pn_kernel,
            grid=(NBLK,),
            in_specs=[_row_spec(F), _STATS_SPEC],
            out_specs=[_row_spec(F), _row_spec(2 * F)],
            out_shape=[jax.ShapeDtypeStruct((C, F), jnp.float32),
                       jax.ShapeDtypeStruct((C, 2 * F), jnp.float32)],
        )(cons_pre, cons_stats)
        p2 = spmm(constraints_xd, cols_idx, rows_idx, vals).reshape(NC, V, F)
        vars_pre, vars_stats = pl.pallas_call(
            _vu_kernel,
            grid=(NBLK,),
            in_specs=[_row_spec(F), _P_SPEC, _full_spec(2 * F, F),
                      _full_spec(1, F), _full_spec(F, F), _full_spec(1, F)],
            out_specs=[_row_spec(F), _STATS_SPEC],
            out_shape=[jax.ShapeDtypeStruct((V, F), jnp.float32),
                       _STATS_SHAPE],
        )(variables, p2, vu_w1, vu_b1.reshape(1, F),
          vu_w2, vu_b2.reshape(1, F))
        variables, variables_xd = pl.pallas_call(
            _pn_kernel,
            grid=(NBLK,),
            in_specs=[_row_spec(F), _STATS_SPEC],
            out_specs=[_row_spec(F), _row_spec(2 * F)],
            out_shape=[jax.ShapeDtypeStruct((V, F), jnp.float32),
                       jax.ShapeDtypeStruct((V, 2 * F), jnp.float32)],
        )(vars_pre, vars_stats)
        bin_o, dec_o = pl.pallas_call(
            _out_kernel,
            grid=(NBLK,),
            in_specs=[_row_spec(F), _STATS_SPEC, _full_spec(F, F),
                      _full_spec(1, F), _full_spec(F, OUT_BITS),
                      _full_spec(1, OUT_BITS), _row_spec(OUT_BITS)],
            out_specs=[_row_spec(OUT_BITS), _row_spec(1)],
            out_shape=[jax.ShapeDtypeStruct((V, OUT_BITS), jnp.float32),
                       jax.ShapeDtypeStruct((V, 1), jnp.float32)],
        )(vars_pre, vars_stats, out_w1, out_b1.reshape(1, F),
          out_w2, out_b2.reshape(1, OUT_BITS), noises[i])
        binary_outputs.append(bin_o)
        decimal_outputs.append(dec_o)

    return (jnp.stack(binary_outputs), jnp.stack(decimal_outputs))
